# baseline (speedup 1.0000x reference)
import jax
import jax.numpy as jnp
from jax import lax
from jax.experimental import pallas as pl
from jax.experimental.pallas import tpu as pltpu
from jax.experimental.pallas import tpu_sc as plsc

_VOCAB = 1000000
_HALF = _VOCAB // 2
_EMBED = 64
_BATCH = 4096
_SEQ = 200
_SEQP = 208
_NC = 2
_NS = 16
_NW = _NC * _NS
_ROWS_PER_W = _BATCH // _NW
_IDX_PER_W = _ROWS_PER_W * _SEQP
_TC_BLK = 4096


def _tw_body(t2_ref, wlo_ref, whi_ref, lo_ref, hi_ref):
    blk = t2_ref[...]
    lo_ref[...] = jnp.sum(blk * wlo_ref[...][:, 0][None, :], axis=1)
    hi_ref[...] = jnp.sum(blk * whi_ref[...][:, 0][None, :], axis=1)


def _table_times_w(t2, wlo, whi):
    return pl.pallas_call(
        _tw_body,
        grid=((_HALF + _TC_BLK - 1) // _TC_BLK,),
        in_specs=[
            pl.BlockSpec((_TC_BLK, 2 * _EMBED), lambda i: (i, 0)),
            pl.BlockSpec((2 * _EMBED, 1), lambda i: (0, 0)),
            pl.BlockSpec((2 * _EMBED, 1), lambda i: (0, 0)),
        ],
        out_specs=[
            pl.BlockSpec((_TC_BLK,), lambda i: (i,)),
            pl.BlockSpec((_TC_BLK,), lambda i: (i,)),
        ],
        out_shape=[
            jax.ShapeDtypeStruct((_HALF,), jnp.float32),
            jax.ShapeDtypeStruct((_HALF,), jnp.float32),
        ],
    )(t2, wlo, whi)


def _sc_body(tw_hbm, xt_hbm, b_hbm, out_hbm, idx_v, vals_v, res_v, b_v, sem):
    wid = lax.axis_index("s") * _NC + lax.axis_index("c")
    base = wid * _ROWS_PER_W

    pltpu.sync_copy(xt_hbm.at[pl.ds(wid * _IDX_PER_W, _IDX_PER_W)], idx_v)
    pltpu.sync_copy(b_hbm, b_v)
    bvec = b_v[...]
    pltpu.async_copy(tw_hbm.at[idx_v], vals_v, sem).wait()

    zero16 = jnp.zeros((16,), jnp.float32)
    for sub in range(_ROWS_PER_W // 16):
        def cbody(c, carry):
            s, cnt = carry
            off = c * _ROWS_PER_W + sub * 16
            v = vals_v[pl.ds(off, 16)]
            ii = idx_v[pl.ds(off, 16)]
            s = s + v
            cnt = cnt + jnp.minimum(ii, 1).astype(jnp.float32)
            return (s, cnt)

        s, cnt = lax.fori_loop(0, _SEQP, cbody, (zero16, zero16))
        z = s / (cnt + 1e-8) + bvec
        res_v[pl.ds(sub * 16, 16)] = 1.0 / (1.0 + jnp.exp(-z))

    pltpu.sync_copy(res_v, out_hbm.at[pl.ds(base, _ROWS_PER_W)])


def _sc_call():
    return pl.kernel(
        _sc_body,
        out_type=jax.ShapeDtypeStruct((_BATCH,), jnp.float32),
        mesh=plsc.VectorSubcoreMesh(
            core_axis_name="c", subcore_axis_name="s",
            num_cores=_NC, num_subcores=_NS),
        scratch_types=[
            pltpu.VMEM((_IDX_PER_W,), jnp.int32),
            pltpu.VMEM((_IDX_PER_W,), jnp.float32),
            pltpu.VMEM((_ROWS_PER_W,), jnp.float32),
            pltpu.VMEM((16,), jnp.float32),
            pltpu.SemaphoreType.DMA,
        ],
    )


def kernel(x, lengths, table, W, b):
    del lengths
    t2 = table.reshape(_HALF, 2 * _EMBED)
    w0 = W[:, 0]
    z64 = jnp.zeros((_EMBED,), jnp.float32)
    wlo = jnp.concatenate([w0, z64]).reshape(2 * _EMBED, 1)
    whi = jnp.concatenate([z64, w0]).reshape(2 * _EMBED, 1)
    tw_lo, tw_hi = _table_times_w(t2, wlo, whi)
    tw_cat = jnp.concatenate([tw_lo, tw_hi])

    xp = jnp.concatenate(
        [x, jnp.zeros((_BATCH, _SEQP - _SEQ), jnp.int32)], axis=1
    )
    xp = (xp & 1) * _HALF + (xp >> 1)
    xt = xp.reshape(_NW, _ROWS_PER_W, _SEQP).transpose(0, 2, 1).reshape(-1)
    b16 = jnp.broadcast_to(b.reshape(()), (16,))
    return _sc_call()(tw_cat, xt, b16)


# speedup vs baseline: 1.6613x; 1.6613x over previous
import jax
import jax.numpy as jnp
from jax import lax
from jax.experimental import pallas as pl
from jax.experimental.pallas import tpu as pltpu
from jax.experimental.pallas import tpu_sc as plsc

_VOCAB = 1000000
_EMBED = 64
_BATCH = 4096
_SEQ = 200
_SEQP = 208
_NC = 2
_NS = 16
_NW = _NC * _NS
_ROWS_PER_W = _BATCH // _NW
_IDX_PER_W = _ROWS_PER_W * _SEQP
_TC_BLK = 8192


def _tw_body(t_ref, w_ref, o_ref):
    o_ref[...] = lax.dot_general(
        w_ref[...], t_ref[...], (((1,), (1,)), ((), ())),
        preferred_element_type=jnp.float32)


def _table_times_w(table, wT):
    return pl.pallas_call(
        _tw_body,
        grid=((_VOCAB + _TC_BLK - 1) // _TC_BLK,),
        in_specs=[
            pl.BlockSpec((_TC_BLK, _EMBED), lambda i: (i, 0)),
            pl.BlockSpec((1, _EMBED), lambda i: (0, 0)),
        ],
        out_specs=pl.BlockSpec((1, _TC_BLK), lambda i: (0, i)),
        out_shape=jax.ShapeDtypeStruct((1, _VOCAB), jnp.float32),
    )(table, wT)


def _sc_body(tw_hbm, xt_hbm, b_hbm, out_hbm, idx_v, vals_v, res_v, b_v, sem):
    wid = lax.axis_index("s") * _NC + lax.axis_index("c")
    base = wid * _ROWS_PER_W

    pltpu.sync_copy(xt_hbm.at[pl.ds(wid * _IDX_PER_W, _IDX_PER_W)], idx_v)
    pltpu.sync_copy(b_hbm, b_v)
    bvec = b_v[...]
    pltpu.async_copy(tw_hbm.at[idx_v], vals_v, sem).wait()

    zero16 = jnp.zeros((16,), jnp.float32)
    for sub in range(_ROWS_PER_W // 16):
        def cbody(c, carry):
            s, cnt = carry
            off = c * _ROWS_PER_W + sub * 16
            v = vals_v[pl.ds(off, 16)]
            ii = idx_v[pl.ds(off, 16)]
            s = s + v
            cnt = cnt + jnp.minimum(ii, 1).astype(jnp.float32)
            return (s, cnt)

        s, cnt = lax.fori_loop(0, _SEQP, cbody, (zero16, zero16))
        z = s / (cnt + 1e-8) + bvec
        res_v[pl.ds(sub * 16, 16)] = 1.0 / (1.0 + jnp.exp(-z))

    pltpu.sync_copy(res_v, out_hbm.at[pl.ds(base, _ROWS_PER_W)])


def _sc_call():
    return pl.kernel(
        _sc_body,
        out_type=jax.ShapeDtypeStruct((_BATCH,), jnp.float32),
        mesh=plsc.VectorSubcoreMesh(
            core_axis_name="c", subcore_axis_name="s",
            num_cores=_NC, num_subcores=_NS),
        scratch_types=[
            pltpu.VMEM((_IDX_PER_W,), jnp.int32),
            pltpu.VMEM((_IDX_PER_W,), jnp.float32),
            pltpu.VMEM((_ROWS_PER_W,), jnp.float32),
            pltpu.VMEM((16,), jnp.float32),
            pltpu.SemaphoreType.DMA,
        ],
    )


def kernel(x, lengths, table, W, b):
    del lengths
    wT = W.reshape(1, _EMBED)
    tw = _table_times_w(table, wT).reshape(_VOCAB)

    xp = jnp.concatenate(
        [x, jnp.zeros((_BATCH, _SEQP - _SEQ), jnp.int32)], axis=1
    )
    xt = xp.reshape(_NW, _ROWS_PER_W, _SEQP).transpose(0, 2, 1).reshape(-1)
    b16 = jnp.broadcast_to(b.reshape(()), (16,))
    return _sc_call()(tw, xt, b16)


# speedup vs baseline: 4.3876x; 2.6412x over previous
import jax
import jax.numpy as jnp
from jax import lax
from jax.experimental import pallas as pl
from jax.experimental.pallas import tpu as pltpu
from jax.experimental.pallas import tpu_sc as plsc

_VOCAB = 1000000
_EMBED = 64
_BATCH = 4096
_SEQ = 200
_SEQP = 208
_NC = 2
_NS = 16
_NW = _NC * _NS
_ROWS_PER_W = _BATCH // _NW
_IDX_PER_W = _ROWS_PER_W * _SEQP
_TC_BLK = 2 ** 15


def _tw_body(t_ref, w_ref, o_ref):
    o = lax.dot_general(
        w_ref[...], t_ref[...], (((1,), (0,)), ((), ())),
        preferred_element_type=jnp.float32)
    o_ref[...] = o[0]


def _table_times_w(tt, wT):
    n_blk = (_VOCAB + _TC_BLK - 1) // _TC_BLK
    return pl.pallas_call(
        _tw_body,
        grid=(n_blk,),
        in_specs=[
            pl.BlockSpec((_EMBED, _TC_BLK), lambda i: (0, i)),
            pl.BlockSpec((1, _EMBED), lambda i: (0, 0)),
        ],
        out_specs=pl.BlockSpec((_TC_BLK,), lambda i: (i,)),
        out_shape=jax.ShapeDtypeStruct((_VOCAB,), jnp.float32),
    )(tt, wT)


def _sc_body(tw_hbm, xt_hbm, b_hbm, out_hbm, idx_v, vals_v, res_v, b_v, sem):
    wid = lax.axis_index("s") * _NC + lax.axis_index("c")
    base = wid * _ROWS_PER_W

    pltpu.sync_copy(xt_hbm.at[pl.ds(wid * _IDX_PER_W, _IDX_PER_W)], idx_v)
    pltpu.sync_copy(b_hbm, b_v)
    bvec = b_v[...]
    pltpu.async_copy(tw_hbm.at[idx_v], vals_v, sem).wait()

    zero16 = jnp.zeros((16,), jnp.float32)
    for sub in range(_ROWS_PER_W // 16):
        def cbody(c, carry):
            s, cnt = carry
            off = c * _ROWS_PER_W + sub * 16
            v = vals_v[pl.ds(off, 16)]
            ii = idx_v[pl.ds(off, 16)]
            s = s + v
            cnt = cnt + jnp.minimum(ii, 1).astype(jnp.float32)
            return (s, cnt)

        s, cnt = lax.fori_loop(0, _SEQP, cbody, (zero16, zero16))
        z = s / (cnt + 1e-8) + bvec
        res_v[pl.ds(sub * 16, 16)] = 1.0 / (1.0 + jnp.exp(-z))

    pltpu.sync_copy(res_v, out_hbm.at[pl.ds(base, _ROWS_PER_W)])


def _sc_call():
    return pl.kernel(
        _sc_body,
        out_type=jax.ShapeDtypeStruct((_BATCH,), jnp.float32),
        mesh=plsc.VectorSubcoreMesh(
            core_axis_name="c", subcore_axis_name="s",
            num_cores=_NC, num_subcores=_NS),
        scratch_types=[
            pltpu.VMEM((_IDX_PER_W,), jnp.int32),
            pltpu.VMEM((_IDX_PER_W,), jnp.float32),
            pltpu.VMEM((_ROWS_PER_W,), jnp.float32),
            pltpu.VMEM((16,), jnp.float32),
            pltpu.SemaphoreType.DMA,
        ],
    )


def kernel(x, lengths, table, W, b):
    del lengths
    tt = table.T
    wT = W.reshape(1, _EMBED)
    tw = _table_times_w(tt, wT)
    import os as _os
    if _os.environ.get("K_PROBE") == "tc_only":
        return tw[:_BATCH]

    xp = jnp.concatenate(
        [x, jnp.zeros((_BATCH, _SEQP - _SEQ), jnp.int32)], axis=1
    )
    xt = xp.reshape(_NW, _ROWS_PER_W, _SEQP).transpose(0, 2, 1).reshape(-1)
    b16 = jnp.broadcast_to(b.reshape(()), (16,))
    return _sc_call()(tw, xt, b16)


# speedup vs baseline: 9.5195x; 2.1696x over previous
import jax
import jax.numpy as jnp
from jax import lax
from jax.experimental import pallas as pl
from jax.experimental.pallas import tpu as pltpu
from jax.experimental.pallas import tpu_sc as plsc

_VOCAB = 1000000
_EMBED = 64
_BATCH = 4096
_SEQ = 200
_SEQP = 208
_NC = 2
_NS = 16
_NW = _NC * _NS
_ROWS_PER_W = _BATCH // _NW
_IDX_PER_W = _ROWS_PER_W * _SEQP
_TC_BLK = 2 ** 15


def _tw_body(t_ref, w_ref, o_ref):
    o = lax.dot_general(
        w_ref[...], t_ref[...], (((1,), (0,)), ((), ())),
        preferred_element_type=jnp.float32)
    o_ref[...] = o[0]


def _table_times_w(tt, wT):
    n_blk = (_VOCAB + _TC_BLK - 1) // _TC_BLK
    return pl.pallas_call(
        _tw_body,
        grid=(n_blk,),
        in_specs=[
            pl.BlockSpec((_EMBED, _TC_BLK), lambda i: (0, i)),
            pl.BlockSpec((1, _EMBED), lambda i: (0, 0)),
        ],
        out_specs=pl.BlockSpec((_TC_BLK,), lambda i: (i,)),
        out_shape=jax.ShapeDtypeStruct((_VOCAB,), jnp.float32),
    )(tt, wT)


def _sc_body(tw_hbm, xt_hbm, b_hbm, out_hbm,
             idx_v, vals_v, res_v, b_v, tw_sh, sem):
    sid = lax.axis_index("s")
    wid = sid * _NC + lax.axis_index("c")
    base = wid * _ROWS_PER_W

    pltpu.sync_copy(xt_hbm.at[pl.ds(wid * _IDX_PER_W, _IDX_PER_W)], idx_v)
    pltpu.sync_copy(b_hbm, b_v)
    bvec = b_v[...]

    @pl.when(sid == 0)
    def _():
        pltpu.sync_copy(tw_hbm, tw_sh)
    plsc.subcore_barrier()

    cp = pltpu.async_copy(tw_sh.at[idx_v], vals_v, sem)

    zero16 = jnp.zeros((16,), jnp.float32)
    cnts = []
    for sub in range(_ROWS_PER_W // 16):
        def cntbody(c, cnt):
            ii = idx_v[pl.ds(c * _ROWS_PER_W + sub * 16, 16)]
            return cnt + jnp.minimum(ii, 1).astype(jnp.float32)
        cnts.append(lax.fori_loop(0, _SEQP, cntbody, zero16))

    cp.wait()
    for sub in range(_ROWS_PER_W // 16):
        def sumbody(c, s):
            return s + vals_v[pl.ds(c * _ROWS_PER_W + sub * 16, 16)]
        s = lax.fori_loop(0, _SEQP, sumbody, zero16)
        z = s / (cnts[sub] + 1e-8) + bvec
        res_v[pl.ds(sub * 16, 16)] = 1.0 / (1.0 + jnp.exp(-z))

    pltpu.sync_copy(res_v, out_hbm.at[pl.ds(base, _ROWS_PER_W)])


def _sc_call():
    return pl.kernel(
        _sc_body,
        out_type=jax.ShapeDtypeStruct((_BATCH,), jnp.float32),
        mesh=plsc.VectorSubcoreMesh(
            core_axis_name="c", subcore_axis_name="s",
            num_cores=_NC, num_subcores=_NS),
        scratch_types=[
            pltpu.VMEM((_IDX_PER_W,), jnp.int32),
            pltpu.VMEM((_IDX_PER_W,), jnp.float32),
            pltpu.VMEM((_ROWS_PER_W,), jnp.float32),
            pltpu.VMEM((16,), jnp.float32),
            pltpu.VMEM_SHARED((_VOCAB,), jnp.float32),
            pltpu.SemaphoreType.DMA,
        ],
    )


def kernel(x, lengths, table, W, b):
    del lengths
    tt = table.T
    wT = W.reshape(1, _EMBED)
    tw = _table_times_w(tt, wT)
    import os as _os
    if _os.environ.get("K_PROBE") == "tc_only":
        return tw[:_BATCH]

    xp = jnp.concatenate(
        [x, jnp.zeros((_BATCH, _SEQP - _SEQ), jnp.int32)], axis=1
    )
    xt = xp.reshape(_NW, _ROWS_PER_W, _SEQP).transpose(0, 2, 1).reshape(-1)
    b16 = jnp.broadcast_to(b.reshape(()), (16,))
    return _sc_call()(tw, xt, b16)
